# baseline (speedup 1.0000x reference)
import functools

import jax
import jax.numpy as jnp
from jax import lax
from jax.experimental import pallas as pl
from jax.experimental.pallas import tpu as pltpu
from jax.experimental.pallas import tpu_sc as plsc

VOCAB = 1000
BATCH = 16384

_NUM_CORES = 2
_NUM_SUBCORES = 16
_NW = _NUM_CORES * _NUM_SUBCORES
_NBLK = VOCAB // 8
_IC = 4096
_NIC = BATCH // _IC


def _make_emb_kernel():
    mesh = plsc.VectorSubcoreMesh(core_axis_name="c", subcore_axis_name="s")

    @functools.partial(
        pl.kernel,
        mesh=mesh,
        out_type=jax.ShapeDtypeStruct((VOCAB, BATCH), jnp.float32),
        compiler_params=pltpu.CompilerParams(
            use_tc_tiling_on_sc=True, needs_layout_passes=False),
        scratch_types=[
            pltpu.VMEM((BATCH,), jnp.int32),
            pltpu.VMEM((8, VOCAB), jnp.float32),
            pltpu.VMEM((8, _IC), jnp.float32),
            pltpu.VMEM((8, _IC), jnp.float32),
            pltpu.SemaphoreType.DMA,
            pltpu.SemaphoreType.DMA,
        ],
    )
    def emb_kernel(idx_hbm, table_t_hbm, out_t_hbm,
                   idx_v, tab_v, ob0, ob1, s0, s1):
        wid = lax.axis_index("s") * _NUM_CORES + lax.axis_index("c")
        pltpu.sync_copy(idx_hbm, idx_v)

        obs = (ob0, ob1)
        ssems = (s0, s1)

        nblocks = jnp.where(wid < _NBLK % _NW, 1 + _NBLK // _NW, _NBLK // _NW)
        ntrip = nblocks * _NIC

        def body(t, _):
            b = wid + _NW * (t // _NIC)
            ic = t % _NIC

            @pl.when(ic == 0)
            def _():
                pltpu.sync_copy(table_t_hbm.at[pl.ds(8 * b, 8)], tab_v)

            for p in range(2):
                ob = obs[p]

                @pl.when((t % 2 == p) & (t >= 2))
                def _():
                    pltpu.make_async_copy(
                        ob, out_t_hbm.at[pl.ds(0, 8), pl.ds(0, _IC)],
                        ssems[p]).wait()

                @pl.when(t % 2 == p)
                def _():
                    @plsc.parallel_loop(0, _IC // 16, unroll=8)
                    def _(s):
                        iv = idx_v[pl.ds(ic * _IC + s * 16, 16)]
                        for r in range(8):
                            rv = jnp.full((16,), r, dtype=jnp.int32)
                            ob[r, pl.ds(s * 16, 16)] = (
                                plsc.load_gather(tab_v, [rv, iv]))
                    pltpu.make_async_copy(
                        ob,
                        out_t_hbm.at[pl.ds(8 * b, 8), pl.ds(ic * _IC, _IC)],
                        ssems[p]).start()
            return 0

        lax.fori_loop(0, ntrip, body, 0)

        for p in range(2):
            pltpu.make_async_copy(
                obs[p], out_t_hbm.at[pl.ds(0, 8), pl.ds(0, _IC)],
                ssems[p]).wait()

    return emb_kernel


_emb_lookup_cache = []


def kernel(idx, embedding_table):
    if not _emb_lookup_cache:
        _emb_lookup_cache.append(_make_emb_kernel())
    idx1 = idx.reshape(-1).astype(jnp.int32)
    return _emb_lookup_cache[0](idx1, embedding_table.T).T


# speedup vs baseline: 1.0925x; 1.0925x over previous
import functools

import jax
import jax.numpy as jnp
from jax import lax
from jax.experimental import pallas as pl
from jax.experimental.pallas import tpu as pltpu
from jax.experimental.pallas import tpu_sc as plsc

VOCAB = 1000
BATCH = 16384

_NUM_CORES = 2
_NUM_SUBCORES = 16
_NW = _NUM_CORES * _NUM_SUBCORES
_NBLK = VOCAB // 8
_IC = 4096
_NIC = BATCH // _IC


def _make_emb_kernel():
    mesh = plsc.VectorSubcoreMesh(core_axis_name="c", subcore_axis_name="s")

    @functools.partial(
        pl.kernel,
        mesh=mesh,
        out_type=jax.ShapeDtypeStruct((VOCAB, BATCH), jnp.float32),
        compiler_params=pltpu.CompilerParams(
            use_tc_tiling_on_sc=True, needs_layout_passes=False),
        scratch_types=[
            pltpu.VMEM((BATCH,), jnp.int32),
            pltpu.VMEM((8, VOCAB), jnp.float32),
            pltpu.VMEM((8, VOCAB), jnp.float32),
            pltpu.VMEM((8, _IC), jnp.float32),
            pltpu.VMEM((8, _IC), jnp.float32),
            pltpu.SemaphoreType.DMA,
            pltpu.SemaphoreType.DMA,
            pltpu.SemaphoreType.DMA,
            pltpu.SemaphoreType.DMA,
        ],
    )
    def emb_kernel(idx_hbm, table_t_hbm, out_t_hbm,
                   idx_v, tb0, tb1, ob0, ob1, s0, s1, ts0, ts1):
        wid = lax.axis_index("s") * _NUM_CORES + lax.axis_index("c")

        obs = (ob0, ob1)
        ssems = (s0, s1)
        tbs = (tb0, tb1)
        tsems = (ts0, ts1)

        nblocks = jnp.where(wid < _NBLK % _NW, 1 + _NBLK // _NW, _NBLK // _NW)
        ntrip = nblocks * _NIC

        pltpu.make_async_copy(
            table_t_hbm.at[pl.ds(8 * wid, 8)], tb0, ts0).start()
        pltpu.sync_copy(idx_hbm, idx_v)

        def body(t, _):
            kb = t // _NIC
            b = wid + _NW * kb
            ic = t % _NIC
            q = kb % 2

            for qq in range(2):
                @pl.when((ic == 0) & (q == qq))
                def _():
                    pltpu.make_async_copy(
                        table_t_hbm.at[pl.ds(8 * b, 8)],
                        tbs[qq], tsems[qq]).wait()

                    @pl.when(kb + 1 < nblocks)
                    def _():
                        pltpu.make_async_copy(
                            table_t_hbm.at[pl.ds(8 * (b + _NW), 8)],
                            tbs[1 - qq], tsems[1 - qq]).start()

            for p in range(2):
                ob = obs[p]

                @pl.when((t % 2 == p) & (t >= 2))
                def _():
                    pltpu.make_async_copy(
                        ob, out_t_hbm.at[pl.ds(0, 8), pl.ds(0, _IC)],
                        ssems[p]).wait()

                for qq in range(2):
                    tab = tbs[qq]

                    @pl.when((t % 2 == p) & (q == qq))
                    def _():
                        @plsc.parallel_loop(0, _IC // 16, unroll=8)
                        def _(s):
                            iv = idx_v[pl.ds(ic * _IC + s * 16, 16)]
                            for r in range(8):
                                rv = jnp.full((16,), r, dtype=jnp.int32)
                                ob[r, pl.ds(s * 16, 16)] = (
                                    plsc.load_gather(tab, [rv, iv]))
                        pltpu.make_async_copy(
                            ob,
                            out_t_hbm.at[pl.ds(8 * b, 8),
                                         pl.ds(ic * _IC, _IC)],
                            ssems[p]).start()
            return 0

        lax.fori_loop(0, ntrip, body, 0)

        for p in range(2):
            pltpu.make_async_copy(
                obs[p], out_t_hbm.at[pl.ds(0, 8), pl.ds(0, _IC)],
                ssems[p]).wait()

    return emb_kernel


_emb_lookup_cache = []


def kernel(idx, embedding_table):
    if not _emb_lookup_cache:
        _emb_lookup_cache.append(_make_emb_kernel())
    idx1 = idx.reshape(-1).astype(jnp.int32)
    return _emb_lookup_cache[0](idx1, embedding_table.T).T
